# baseline (speedup 1.0000x reference)
import functools

import jax
import jax.numpy as jnp
from jax import lax
from jax.experimental import pallas as pl
from jax.experimental.pallas import tpu as pltpu
from jax.experimental.pallas import tpu_sc as plsc

BATCH = 16384
N_FIELDS = 26
VOCAB = 100000
EMBED = 16
N_CONTI = 74
OUT_W = N_FIELDS * EMBED + N_CONTI

NC, NS, L = 2, 16, 16
NW = NC * NS
ROWS_PER_W = BATCH // NW
C = 128
N_CHUNKS = ROWS_PER_W // C
G = C // L


def _body(x_hbm, w_hbm, out_hbm, x_buf, idx_buf, emb_buf, conti_buf,
          gsem, ssem):
    wid = lax.axis_index("s") * NC + lax.axis_index("c")
    iota = lax.iota(jnp.int32, L)

    def chunk(t, _):
        rowbase = wid * ROWS_PER_W + t * C
        pltpu.sync_copy(x_hbm.at[pl.ds(rowbase, C)], x_buf)

        def field(f, _):
            fcol = jnp.full((L,), f, jnp.int32)
            base = f * VOCAB
            for g in range(G):
                b_vec = iota + g * L
                codes = plsc.load_gather(x_buf, [b_vec, fcol])
                idx_buf[f, pl.ds(g * L, L)] = codes + base
            pltpu.async_copy(
                w_hbm.at[idx_buf.at[f]],
                emb_buf.at[pl.ds(pl.multiple_of(f * C, C), C)],
                gsem,
            )
            return 0

        lax.fori_loop(0, N_FIELDS, field, 0)

        def conti(j, _):
            src_col = jnp.full((L,), j + N_FIELDS, jnp.int32)
            dst_col = jnp.full((L,), j, jnp.int32)
            for g in range(G):
                b_vec = iota + g * L
                vals = plsc.load_gather(x_buf, [b_vec, src_col])
                plsc.store_scatter(conti_buf, [b_vec, dst_col],
                                   vals.astype(jnp.float32))
            return 0

        lax.fori_loop(0, N_CONTI, conti, 0)

        def drain(f, _):
            pltpu.make_async_copy(
                w_hbm.at[idx_buf.at[f]],
                emb_buf.at[pl.ds(pl.multiple_of(f * C, C), C)],
                gsem,
            ).wait()
            return 0

        lax.fori_loop(0, N_FIELDS, drain, 0)

        def store(f, _):
            pltpu.async_copy(
                emb_buf.at[pl.ds(pl.multiple_of(f * C, C), C)],
                out_hbm.at[pl.ds(rowbase, C), pl.ds(f * EMBED, EMBED)],
                ssem,
            )
            return 0

        lax.fori_loop(0, N_FIELDS, store, 0)
        pltpu.async_copy(
            conti_buf, out_hbm.at[pl.ds(rowbase, C), pl.ds(N_FIELDS * EMBED, N_CONTI)],
            ssem,
        )

        def drain_store(f, _):
            pltpu.make_async_copy(
                emb_buf.at[pl.ds(pl.multiple_of(f * C, C), C)],
                out_hbm.at[pl.ds(rowbase, C), pl.ds(f * EMBED, EMBED)],
                ssem,
            ).wait()
            return 0

        lax.fori_loop(0, N_FIELDS, drain_store, 0)
        pltpu.make_async_copy(
            conti_buf, out_hbm.at[pl.ds(rowbase, C), pl.ds(N_FIELDS * EMBED, N_CONTI)],
            ssem,
        ).wait()
        return 0

    lax.fori_loop(0, N_CHUNKS, chunk, 0)


@functools.partial(jax.jit, static_argnames=())
def kernel(x, W):
    w_flat = W.reshape(N_FIELDS * VOCAB, EMBED)
    run = functools.partial(
        pl.kernel,
        out_type=jax.ShapeDtypeStruct((BATCH, OUT_W), jnp.float32),
        mesh=plsc.VectorSubcoreMesh(core_axis_name="c", subcore_axis_name="s"),
        compiler_params=pltpu.CompilerParams(
            use_tc_tiling_on_sc=False, needs_layout_passes=False),
        scratch_types=[
            pltpu.VMEM((C, 100), jnp.int32),
            pltpu.VMEM((N_FIELDS, C), jnp.int32),
            pltpu.VMEM((N_FIELDS * C, EMBED), jnp.float32),
            pltpu.VMEM((C, N_CONTI), jnp.float32),
            pltpu.SemaphoreType.DMA,
            pltpu.SemaphoreType.DMA,
        ],
    )(_body)
    return run(x, w_flat)
